# baseline (speedup 1.0000x reference)
import functools

import jax
import jax.numpy as jnp
from jax import lax
from jax.experimental import pallas as pl
from jax.experimental.pallas import tpu as pltpu
from jax.experimental.pallas import tpu_sc as plsc

BATCH = 16384
D = 64
NW = 32
BPW = BATCH // NW
CH = 128
NCH = BPW // CH
PROJ_BLK = 16384


def _proj2_body(nu, xu_ref, xm_ref, w_ref, b_ref, ou_ref, om_ref):
    j = pl.program_id(0)

    @pl.when(j < nu)
    def _():
        r = lax.dot_general(
            w_ref[:, :D], xu_ref[...], (((1,), (0,)), ((), ())),
            preferred_element_type=jnp.float32)
        ou_ref[...] = r[0] + b_ref[0]

    @pl.when(j >= nu)
    def _():
        r = lax.dot_general(
            w_ref[:, D:], xm_ref[...], (((1,), (0,)), ((), ())),
            preferred_element_type=jnp.float32)
        om_ref[...] = r[0]


def _project2(utT, mtT, wT, b):
    nu_blocks = (utT.shape[1] + PROJ_BLK - 1) // PROJ_BLK
    nm_blocks = (mtT.shape[1] + PROJ_BLK - 1) // PROJ_BLK
    grid = nu_blocks + nm_blocks
    last_u = nu_blocks - 1
    last_m = nm_blocks - 1
    body = functools.partial(_proj2_body, nu_blocks)
    return pl.pallas_call(
        body,
        grid=(grid,),
        in_specs=[
            pl.BlockSpec((D, PROJ_BLK), lambda j: (0, jnp.minimum(j, last_u))),
            pl.BlockSpec((D, PROJ_BLK),
                         lambda j: (0, jnp.clip(j - nu_blocks, 0, last_m))),
            pl.BlockSpec((1, 2 * D), lambda j: (0, 0)),
            pl.BlockSpec(memory_space=pltpu.SMEM),
        ],
        out_specs=[
            pl.BlockSpec((PROJ_BLK,), lambda j: (jnp.minimum(j, last_u),)),
            pl.BlockSpec((PROJ_BLK,),
                         lambda j: (jnp.clip(j - nu_blocks, 0, last_m),)),
        ],
        out_shape=[
            jax.ShapeDtypeStruct((utT.shape[1],), jnp.float32),
            jax.ShapeDtypeStruct((mtT.shape[1],), jnp.float32),
        ],
    )(utT, mtT, wT, b)


def _sc_body(uids, mids, uproj, mproj, out,
             idx_u, idx_m, uvals, mvals, outv, sem):
    cid = lax.axis_index("c")
    sid = lax.axis_index("s")
    wid = sid * 2 + cid
    base = wid * BPW

    pltpu.sync_copy(uids.at[pl.ds(base, BPW)], idx_u)
    pltpu.sync_copy(mids.at[pl.ds(base, BPW)], idx_m)

    copies = []
    for c in range(NCH):
        copies.append(pltpu.async_copy(
            uproj.at[idx_u.at[pl.ds(c * CH, CH)]],
            uvals.at[pl.ds(c * CH, CH)], sem.at[c]))
        copies.append(pltpu.async_copy(
            mproj.at[idx_m.at[pl.ds(c * CH, CH)]],
            mvals.at[pl.ds(c * CH, CH)], sem.at[c]))

    for c in range(NCH):
        copies[2 * c].wait()
        copies[2 * c + 1].wait()
        for g in range(CH // 16):
            o = c * CH + g * 16
            outv[pl.ds(o, 16)] = uvals[pl.ds(o, 16)] + mvals[pl.ds(o, 16)]

    pltpu.sync_copy(outv, out.at[pl.ds(base, BPW)])


@jax.jit
def _cf(uids2, mids2, user_tT, movie_tT, wT, b):
    u_proj, m_proj = _project2(user_tT, movie_tT, wT, b)
    mesh = plsc.VectorSubcoreMesh(core_axis_name="c", subcore_axis_name="s")
    kern = functools.partial(
        pl.kernel,
        out_type=jax.ShapeDtypeStruct((BATCH,), jnp.float32),
        mesh=mesh,
        compiler_params=pltpu.CompilerParams(
            needs_layout_passes=False, use_tc_tiling_on_sc=False),
        scratch_types=[
            pltpu.VMEM((BPW,), jnp.int32),
            pltpu.VMEM((BPW,), jnp.int32),
            pltpu.VMEM((BPW,), jnp.float32),
            pltpu.VMEM((BPW,), jnp.float32),
            pltpu.VMEM((BPW,), jnp.float32),
            pltpu.SemaphoreType.DMA((NCH,)),
        ],
    )(_sc_body)
    return kern(uids2, mids2, u_proj, m_proj)


def kernel(user_ids, movie_ids, user_table, movie_table, W, b):
    return _cf(user_ids.astype(jnp.int32), movie_ids.astype(jnp.int32),
               user_table.T, movie_table.T, W.T.astype(jnp.float32),
               b.astype(jnp.float32))


# speedup vs baseline: 1.1054x; 1.1054x over previous
import functools

import jax
import jax.numpy as jnp
from jax import lax
from jax.experimental import pallas as pl
from jax.experimental.pallas import tpu as pltpu
from jax.experimental.pallas import tpu_sc as plsc

BATCH = 16384
D = 64
NW = 32
BPW = BATCH // NW
CH = 128
NCH = BPW // CH
PROJ_BLK = 49152


def _proj2_body(nu, xu_ref, xm_ref, w_ref, b_ref, ou_ref, om_ref):
    j = pl.program_id(0)

    @pl.when(j < nu)
    def _():
        r = lax.dot_general(
            w_ref[:, :D], xu_ref[...], (((1,), (0,)), ((), ())),
            preferred_element_type=jnp.float32)
        ou_ref[...] = r[0] + b_ref[0]

    @pl.when(j >= nu)
    def _():
        r = lax.dot_general(
            w_ref[:, D:], xm_ref[...], (((1,), (0,)), ((), ())),
            preferred_element_type=jnp.float32)
        om_ref[...] = r[0]


def _project2(utT, mtT, wT, b):
    nu_blocks = (utT.shape[1] + PROJ_BLK - 1) // PROJ_BLK
    nm_blocks = (mtT.shape[1] + PROJ_BLK - 1) // PROJ_BLK
    grid = nu_blocks + nm_blocks
    last_u = nu_blocks - 1
    last_m = nm_blocks - 1
    body = functools.partial(_proj2_body, nu_blocks)
    return pl.pallas_call(
        body,
        grid=(grid,),
        in_specs=[
            pl.BlockSpec((D, PROJ_BLK), lambda j: (0, jnp.minimum(j, last_u))),
            pl.BlockSpec((D, PROJ_BLK),
                         lambda j: (0, jnp.clip(j - nu_blocks, 0, last_m))),
            pl.BlockSpec((1, 2 * D), lambda j: (0, 0)),
            pl.BlockSpec(memory_space=pltpu.SMEM),
        ],
        out_specs=[
            pl.BlockSpec((PROJ_BLK,), lambda j: (jnp.minimum(j, last_u),)),
            pl.BlockSpec((PROJ_BLK,),
                         lambda j: (jnp.clip(j - nu_blocks, 0, last_m),)),
        ],
        out_shape=[
            jax.ShapeDtypeStruct((utT.shape[1],), jnp.float32),
            jax.ShapeDtypeStruct((mtT.shape[1],), jnp.float32),
        ],
    )(utT, mtT, wT, b)


def _sc_body(uids, mids, uproj, mproj, out,
             idx_u, idx_m, uvals, mvals, outv, sem):
    cid = lax.axis_index("c")
    sid = lax.axis_index("s")
    wid = sid * 2 + cid
    base = wid * BPW

    pltpu.sync_copy(uids.at[pl.ds(base, BPW)], idx_u)
    pltpu.sync_copy(mids.at[pl.ds(base, BPW)], idx_m)

    copies = []
    for c in range(NCH):
        copies.append(pltpu.async_copy(
            uproj.at[idx_u.at[pl.ds(c * CH, CH)]],
            uvals.at[pl.ds(c * CH, CH)], sem.at[c]))
        copies.append(pltpu.async_copy(
            mproj.at[idx_m.at[pl.ds(c * CH, CH)]],
            mvals.at[pl.ds(c * CH, CH)], sem.at[c]))

    for c in range(NCH):
        copies[2 * c].wait()
        copies[2 * c + 1].wait()
        for g in range(CH // 16):
            o = c * CH + g * 16
            outv[pl.ds(o, 16)] = uvals[pl.ds(o, 16)] + mvals[pl.ds(o, 16)]

    pltpu.sync_copy(outv, out.at[pl.ds(base, BPW)])


@jax.jit
def _cf(uids2, mids2, user_tT, movie_tT, wT, b):
    u_proj, m_proj = _project2(user_tT, movie_tT, wT, b)
    mesh = plsc.VectorSubcoreMesh(core_axis_name="c", subcore_axis_name="s")
    kern = functools.partial(
        pl.kernel,
        out_type=jax.ShapeDtypeStruct((BATCH,), jnp.float32),
        mesh=mesh,
        compiler_params=pltpu.CompilerParams(
            needs_layout_passes=False, use_tc_tiling_on_sc=False),
        scratch_types=[
            pltpu.VMEM((BPW,), jnp.int32),
            pltpu.VMEM((BPW,), jnp.int32),
            pltpu.VMEM((BPW,), jnp.float32),
            pltpu.VMEM((BPW,), jnp.float32),
            pltpu.VMEM((BPW,), jnp.float32),
            pltpu.SemaphoreType.DMA((NCH,)),
        ],
    )(_sc_body)
    return kern(uids2, mids2, u_proj, m_proj)


def kernel(user_ids, movie_ids, user_table, movie_table, W, b):
    return _cf(user_ids.astype(jnp.int32), movie_ids.astype(jnp.int32),
               user_table.T, movie_table.T, W.T.astype(jnp.float32),
               b.astype(jnp.float32))


# speedup vs baseline: 1.1232x; 1.0161x over previous
import functools

import jax
import jax.numpy as jnp
from jax import lax
from jax.experimental import pallas as pl
from jax.experimental.pallas import tpu as pltpu
from jax.experimental.pallas import tpu_sc as plsc

BATCH = 16384
D = 64
NW = 32
BPW = BATCH // NW
CH = 128
NCH = BPW // CH
PROJ_BLK = 32768


def _proj2_body(nu, xu_ref, xm_ref, w_ref, b_ref, ou_ref, om_ref):
    j = pl.program_id(0)

    @pl.when(j < nu)
    def _():
        r = lax.dot_general(
            w_ref[:, :D], xu_ref[...], (((1,), (0,)), ((), ())),
            preferred_element_type=jnp.float32)
        ou_ref[...] = r[0] + b_ref[0]

    @pl.when(j >= nu)
    def _():
        r = lax.dot_general(
            w_ref[:, D:], xm_ref[...], (((1,), (0,)), ((), ())),
            preferred_element_type=jnp.float32)
        om_ref[...] = r[0]


def _project2(utT, mtT, wT, b):
    nu_blocks = (utT.shape[1] + PROJ_BLK - 1) // PROJ_BLK
    nm_blocks = (mtT.shape[1] + PROJ_BLK - 1) // PROJ_BLK
    grid = nu_blocks + nm_blocks
    last_u = nu_blocks - 1
    last_m = nm_blocks - 1
    body = functools.partial(_proj2_body, nu_blocks)
    return pl.pallas_call(
        body,
        grid=(grid,),
        in_specs=[
            pl.BlockSpec((D, PROJ_BLK), lambda j: (0, jnp.minimum(j, last_u))),
            pl.BlockSpec((D, PROJ_BLK),
                         lambda j: (0, jnp.clip(j - nu_blocks, 0, last_m))),
            pl.BlockSpec((1, 2 * D), lambda j: (0, 0)),
            pl.BlockSpec(memory_space=pltpu.SMEM),
        ],
        out_specs=[
            pl.BlockSpec((PROJ_BLK,), lambda j: (jnp.minimum(j, last_u),)),
            pl.BlockSpec((PROJ_BLK,),
                         lambda j: (jnp.clip(j - nu_blocks, 0, last_m),)),
        ],
        out_shape=[
            jax.ShapeDtypeStruct((utT.shape[1],), jnp.float32),
            jax.ShapeDtypeStruct((mtT.shape[1],), jnp.float32),
        ],
    )(utT, mtT, wT, b)


def _sc_body(uids, mids, uproj, mproj, out,
             idx_u, idx_m, uvals, mvals, outv, sem):
    cid = lax.axis_index("c")
    sid = lax.axis_index("s")
    wid = sid * 2 + cid
    base = wid * BPW

    pltpu.sync_copy(uids.at[pl.ds(base, BPW)], idx_u)
    pltpu.sync_copy(mids.at[pl.ds(base, BPW)], idx_m)

    copies = []
    for c in range(NCH):
        copies.append(pltpu.async_copy(
            uproj.at[idx_u.at[pl.ds(c * CH, CH)]],
            uvals.at[pl.ds(c * CH, CH)], sem.at[c]))
        copies.append(pltpu.async_copy(
            mproj.at[idx_m.at[pl.ds(c * CH, CH)]],
            mvals.at[pl.ds(c * CH, CH)], sem.at[c]))

    for c in range(NCH):
        copies[2 * c].wait()
        copies[2 * c + 1].wait()
        for g in range(CH // 16):
            o = c * CH + g * 16
            outv[pl.ds(o, 16)] = uvals[pl.ds(o, 16)] + mvals[pl.ds(o, 16)]

    pltpu.sync_copy(outv, out.at[pl.ds(base, BPW)])


@jax.jit
def _cf(uids2, mids2, user_tT, movie_tT, wT, b):
    u_proj, m_proj = _project2(user_tT, movie_tT, wT, b)
    mesh = plsc.VectorSubcoreMesh(core_axis_name="c", subcore_axis_name="s")
    kern = functools.partial(
        pl.kernel,
        out_type=jax.ShapeDtypeStruct((BATCH,), jnp.float32),
        mesh=mesh,
        compiler_params=pltpu.CompilerParams(
            needs_layout_passes=False, use_tc_tiling_on_sc=False),
        scratch_types=[
            pltpu.VMEM((BPW,), jnp.int32),
            pltpu.VMEM((BPW,), jnp.int32),
            pltpu.VMEM((BPW,), jnp.float32),
            pltpu.VMEM((BPW,), jnp.float32),
            pltpu.VMEM((BPW,), jnp.float32),
            pltpu.SemaphoreType.DMA((NCH,)),
        ],
    )(_sc_body)
    return kern(uids2, mids2, u_proj, m_proj)


def kernel(user_ids, movie_ids, user_table, movie_table, W, b):
    return _cf(user_ids.astype(jnp.int32), movie_ids.astype(jnp.int32),
               user_table.T, movie_table.T, W.T.astype(jnp.float32),
               b.astype(jnp.float32))
